# baseline (speedup 1.0000x reference)
import functools

import jax
import jax.numpy as jnp
from jax import lax
from jax.experimental import pallas as pl
from jax.experimental.pallas import tpu as pltpu

_EPS = 1e-5


def _pick_block(n):
    for blk in (5000, 4096, 4000, 2500, 2048, 2000, 1024, 1000):
        if n % blk == 0:
            return blk
    return n


def _colstats(m):
    return jnp.concatenate(
        [jnp.sum(m, axis=0, keepdims=True),
         jnp.sum(m * m, axis=0, keepdims=True)], axis=0)[None]


def _bn_affine(partials, g, be, inv_n):
    tot = jnp.sum(partials, axis=0)
    mean = tot[0:1] * inv_n
    var = tot[1:2] * inv_n - mean * mean
    a = g * lax.rsqrt(var + _EPS)
    c = be - mean * a
    return a, c


def _stats_x_kernel(x_ref, out_ref):
    out_ref[...] = _colstats(x_ref[...])


def _hidden(x_ref, W1_ref, b1_ref, sA_ref, g1_ref, be1_ref, inv_n):
    a1, c1 = _bn_affine(sA_ref[...], g1_ref[...], be1_ref[...], inv_n)
    xs = x_ref[...] * a1 + c1
    z = lax.dot_general(xs, W1_ref[...], (((1,), (1,)), ((), ())),
                        preferred_element_type=jnp.float32)
    return jnp.maximum(z + b1_ref[...], 0.0)


def _stats_h_kernel(x_ref, W1_ref, b1_ref, g1_ref, be1_ref, sA_ref, out_ref,
                    *, inv_n):
    h = _hidden(x_ref, W1_ref, b1_ref, sA_ref, g1_ref, be1_ref, inv_n)
    out_ref[...] = _colstats(h)


def _final_kernel(x_ref, W1_ref, b1_ref, g1_ref, be1_ref, sA_ref,
                  W2_ref, b2_ref, g2_ref, be2_ref, sB_ref, out_ref, *, inv_n):
    h = _hidden(x_ref, W1_ref, b1_ref, sA_ref, g1_ref, be1_ref, inv_n)
    a2, c2 = _bn_affine(sB_ref[...], g2_ref[...], be2_ref[...], inv_n)
    hs = h * a2 + c2
    out = lax.dot_general(hs, W2_ref[...], (((1,), (1,)), ((), ())),
                          preferred_element_type=jnp.float32)
    out_ref[...] = out + b2_ref[...]


def kernel(causal, gamma1, beta1, W1, b1, gamma2, beta2, W2, b2):
    n, d = causal.shape
    d_out = W2.shape[0]
    blk = _pick_block(n)
    nb = n // blk
    inv_n = 1.0 / n

    row = lambda v: v.reshape(1, -1)

    def full(shape):
        return pl.BlockSpec(shape, lambda i: (0,) * len(shape))

    x_spec = pl.BlockSpec((blk, d), lambda i: (i, 0))
    pstat_out_spec = pl.BlockSpec((1, 2, d), lambda i: (i, 0, 0))
    pstat_shape = jax.ShapeDtypeStruct((nb, 2, d), jnp.float32)
    parallel = pltpu.CompilerParams(dimension_semantics=("parallel",))

    stats_x = pl.pallas_call(
        _stats_x_kernel,
        grid=(nb,),
        in_specs=[x_spec],
        out_specs=pstat_out_spec,
        out_shape=pstat_shape,
        compiler_params=parallel,
    )(causal)

    stats_h = pl.pallas_call(
        functools.partial(_stats_h_kernel, inv_n=inv_n),
        grid=(nb,),
        in_specs=[x_spec, full((d, d)), full((1, d)), full((1, d)),
                  full((1, d)), full((nb, 2, d))],
        out_specs=pstat_out_spec,
        out_shape=pstat_shape,
        compiler_params=parallel,
    )(causal, W1, row(b1), row(gamma1), row(beta1), stats_x)

    out = pl.pallas_call(
        functools.partial(_final_kernel, inv_n=inv_n),
        grid=(nb,),
        in_specs=[x_spec, full((d, d)), full((1, d)), full((1, d)),
                  full((1, d)), full((nb, 2, d)), full((d_out, d)),
                  full((1, d_out)), full((1, d)), full((1, d)),
                  full((nb, 2, d))],
        out_specs=pl.BlockSpec((blk, d_out), lambda i: (i, 0)),
        out_shape=jax.ShapeDtypeStruct((n, d_out), jnp.float32),
        compiler_params=parallel,
    )(causal, W1, row(b1), row(gamma1), row(beta1), stats_x,
      W2, row(b2), row(gamma2), row(beta2), stats_h)

    return out
